# baseline (speedup 1.0000x reference)
import functools

import jax
import jax.numpy as jnp
from jax import lax
from jax.experimental import pallas as pl
from jax.experimental.pallas import tpu as pltpu
from jax.experimental.pallas import tpu_sc as plsc

N = 10000
D = 128
H = 64
OUT = 32

NC = 2
NS = 16
NW = NC * NS
E = 320000
EW = E // NW
C = 48
CH = 209
EWP = CH * C
GW = 256
OW = 128
NBLK = OW // 16
STRIPE = 672

_mesh = plsc.VectorSubcoreMesh(core_axis_name="c", subcore_axis_name="s")


def _proj_kernel(x_ref, m_ref, xw_ref, ptab_ref, xv_ref):
    p = jnp.dot(x_ref[...], m_ref[...], preferred_element_type=jnp.float32)
    xw_ref[...] = p[:, :H]
    ptab_ref[...] = p[:, H:H + GW]
    xv_ref[...] = p[:, H + GW:]


def _mid_kernel(acc_ref, xv_ref, tab_ref):
    g = acc_ref[0] + acc_ref[1]
    xv = xv_ref[...]
    tab_ref[...] = jnp.concatenate(
        [g[:, :H], xv[:, :H], g[:, H:], xv[:, H:]], axis=1)


def _final_kernel(xw_ref, acc2_ref, b1_ref, w2_ref, b2_ref, out_ref):
    s = acc2_ref[0] + acc2_ref[1]
    pre = xw_ref[...] + s[:, :H] + s[:, H:] + b1_ref[...]
    out_ref[...] = (
        jnp.dot(jnp.maximum(pre, 0.0), w2_ref[...],
                preferred_element_type=jnp.float32)
        + b2_ref[...]
    )


@functools.partial(
    pl.kernel,
    out_type=jax.ShapeDtypeStruct((NC * N, OW), jnp.float32),
    mesh=_mesh,
    scratch_types=[
        pltpu.VMEM((4, 4, C), jnp.int32),
        pltpu.VMEM((2, C, GW), jnp.float32),
        pltpu.VMEM((2, C, OW), jnp.float32),
        pltpu.VMEM_SHARED((N, OW), jnp.float32),
        pltpu.SemaphoreType.DMA,
        pltpu.SemaphoreType.DMA,
        pltpu.SemaphoreType.DMA,
        pltpu.SemaphoreType.DMA,
        pltpu.SemaphoreType.DMA,
        pltpu.SemaphoreType.DMA,
        pltpu.SemaphoreType.DMA,
        pltpu.SemaphoreType.DMA,
    ],
)
def _sc_stage(tab_hbm, meta_hbm, out_hbm, meta_v, rows_v, orow_v, acc_sh,
              m0, m1, m2, m3, g0, g1, s0, s1):
    msem = (m0, m1, m2, m3)
    gsem = (g0, g1)
    ssem = (s0, s1)
    cid = lax.axis_index("c")
    sid = lax.axis_index("s")
    wid = cid * NS + sid
    mbase = wid * CH

    def meta_issue(i, slot):
        pltpu.async_copy(meta_hbm.at[mbase + i], meta_v.at[slot], msem[slot])

    def meta_wait(slot):
        pltpu.make_async_copy(
            meta_hbm.at[mbase], meta_v.at[slot], msem[slot]).wait()

    CH2 = C // 2

    def gather_issue(slot, b):
        idx = meta_v.at[slot].at[0]
        pltpu.async_copy(
            tab_hbm.at[idx.at[pl.ds(0, CH2)]],
            rows_v.at[b, pl.ds(0, CH2)], gsem[b])
        pltpu.async_copy(
            tab_hbm.at[idx.at[pl.ds(CH2, CH2)]],
            rows_v.at[b, pl.ds(CH2, CH2)], gsem[b])

    def gather_wait(slot, b):
        idx = meta_v.at[slot].at[0]
        pltpu.make_async_copy(
            tab_hbm.at[idx.at[pl.ds(0, CH2)]],
            rows_v.at[b, pl.ds(0, CH2)], gsem[b]).wait()
        pltpu.make_async_copy(
            tab_hbm.at[idx.at[pl.ds(CH2, CH2)]],
            rows_v.at[b, pl.ds(CH2, CH2)], gsem[b]).wait()

    def scatter_issue(slot, b):
        pltpu.async_copy(
            orow_v.at[b],
            acc_sh.at[meta_v.at[slot].at[1]], ssem[b], add=True)

    def scatter_wait(slot, b):
        pltpu.make_async_copy(
            orow_v.at[b],
            acc_sh.at[meta_v.at[slot].at[1]], ssem[b]).wait()

    def compute(slot, b):
        def grp(g, _):
            w0f = lax.bitcast_convert_type(meta_v[slot, 2, pl.ds(g * 16, 16)],
                                           jnp.float32)
            w1f = lax.bitcast_convert_type(meta_v[slot, 3, pl.ds(g * 16, 16)],
                                           jnp.float32)

            def edge(e16, _):
                e = g * 16 + e16
                lane = jnp.full((16,), e16, jnp.int32)
                w0b = w0f[lane]
                w1b = w1f[lane]
                for c in range(NBLK):
                    a = rows_v[b, e, pl.ds(c * 16, 16)]
                    bb = rows_v[b, e, pl.ds(OW + c * 16, 16)]
                    orow_v[b, e, pl.ds(c * 16, 16)] = a * w0b + bb * w1b
                return 0

            lax.fori_loop(0, 16, edge, 0)
            return 0

        lax.fori_loop(0, C // 16, grp, 0)

    zv = jnp.zeros((16,), jnp.float32)

    def zrow(e, _):
        for c in range(NBLK):
            orow_v[0, e, pl.ds(c * 16, 16)] = zv
        return 0

    lax.fori_loop(0, C, zrow, 0)
    s625 = sid * (N // NS)
    start = jnp.minimum((s625 // 8) * 8, N - STRIPE)
    start = pl.multiple_of(start, 8)
    for j in range(STRIPE // C):
        pltpu.sync_copy(orow_v.at[0],
                        acc_sh.at[pl.ds(start + j * C, C)])
    plsc.subcore_barrier()

    meta_issue(0, 0)
    meta_issue(1, 1)
    meta_wait(0)
    gather_issue(0, 0)

    meta_wait(1)
    gather_issue(1, 1)
    meta_issue(2, 2)
    gather_wait(0, 0)
    compute(0, 0)
    scatter_issue(0, 0)

    meta_wait(2)
    gather_issue(2, 0)
    meta_issue(3, 3)
    gather_wait(1, 1)
    compute(1, 1)
    scatter_issue(1, 1)

    def body(gi, _):
        i0 = 4 * gi + 2
        for k in range(4):
            i = i0 + k
            slot = (2 + k) % 4
            b = (2 + k) % 2
            nslot = (slot + 1) % 4
            fslot = (slot + 2) % 4
            meta_wait(nslot)
            gather_issue(nslot, b ^ 1)
            scatter_wait(fslot, b)
            meta_issue(i + 2, fslot)
            gather_wait(slot, b)
            compute(slot, b)
            scatter_issue(slot, b)
        return 0

    lax.fori_loop(0, 51, body, 0)

    meta_wait(3)
    gather_issue(3, 1)
    scatter_wait(0, 0)
    meta_issue(208, 0)
    gather_wait(2, 0)
    compute(2, 0)
    scatter_issue(2, 0)

    meta_wait(0)
    gather_issue(0, 0)
    scatter_wait(1, 1)
    gather_wait(3, 1)
    compute(3, 1)
    scatter_issue(3, 1)

    scatter_wait(2, 0)
    gather_wait(0, 0)
    compute(0, 0)
    scatter_issue(0, 0)

    scatter_wait(3, 1)
    scatter_wait(0, 0)
    plsc.subcore_barrier()

    pltpu.sync_copy(
        acc_sh.at[pl.ds(start, STRIPE)],
        out_hbm.at[pl.ds(cid * N + start, STRIPE)])


def kernel(x, edge_index, kernel_w, W1, b1, W2, b2):
    W1x = W1[0:D]
    V0, V1 = W1[D:2 * D], W1[2 * D:3 * D]
    U00, U01 = W1[3 * D:4 * D], W1[4 * D:5 * D]
    U10, U11 = W1[5 * D:6 * D], W1[6 * D:7 * D]
    M = jnp.concatenate([W1x, U00, U10, U01, U11, V0, V1], axis=1)

    def _pad_worker(a):
        aw = a.reshape(NW, EW)
        return jnp.pad(aw, ((0, 0), (0, EWP - EW))).reshape(NW * CH, C)

    meta = jnp.stack(
        [_pad_worker(edge_index[0]),
         _pad_worker(edge_index[1]),
         _pad_worker(lax.bitcast_convert_type(kernel_w[0], jnp.int32)),
         _pad_worker(lax.bitcast_convert_type(kernel_w[1], jnp.int32))],
        axis=1)

    xw1x, ptab, xv = pl.pallas_call(
        _proj_kernel,
        out_shape=[jax.ShapeDtypeStruct((N, H), jnp.float32),
                   jax.ShapeDtypeStruct((N, GW), jnp.float32),
                   jax.ShapeDtypeStruct((N, OW), jnp.float32)],
    )(x, M)

    acc1 = _sc_stage(ptab, meta).reshape(NC, N, OW)

    tab2 = pl.pallas_call(
        _mid_kernel,
        out_shape=jax.ShapeDtypeStruct((N, GW), jnp.float32),
    )(acc1, xv)

    acc2 = _sc_stage(tab2, meta).reshape(NC, N, OW)

    emb = pl.pallas_call(
        _final_kernel,
        out_shape=jax.ShapeDtypeStruct((N, OUT), jnp.float32),
    )(xw1x, acc2, b1.reshape(1, H), W2, b2.reshape(1, OUT))
    return emb


# speedup vs baseline: 1.0993x; 1.0993x over previous
import functools

import jax
import jax.numpy as jnp
from jax import lax
from jax.experimental import pallas as pl
from jax.experimental.pallas import tpu as pltpu
from jax.experimental.pallas import tpu_sc as plsc

N = 10000
D = 128
H = 64
OUT = 32

NC = 2
NS = 16
NW = NC * NS
E = 320000
EW = E // NW
C = 80
CH = EW // C
GW = 256
GWI = 128
OW = 128
NBLK = OW // 16
STRIPE = 640

_mesh = plsc.VectorSubcoreMesh(core_axis_name="c", subcore_axis_name="s")


def _pack_bf16_pair(a, b):
    ai = lax.bitcast_convert_type(a.astype(jnp.bfloat16), jnp.int16)
    bi = lax.bitcast_convert_type(b.astype(jnp.bfloat16), jnp.int16)
    return (ai.astype(jnp.int32) & 0xFFFF) | (bi.astype(jnp.int32) << 16)


def _proj_kernel(x_ref, m_ref, xw_ref, ptab_ref, xv_ref):
    p = jnp.dot(x_ref[...], m_ref[...], preferred_element_type=jnp.float32)
    xw_ref[...] = p[:, :H]
    ptab_ref[...] = _pack_bf16_pair(p[:, H:H + GWI], p[:, H + GWI:H + GW])
    xv_ref[...] = p[:, H + GW:]


def _mid_kernel(acc_ref, xv_ref, tab_ref):
    g = acc_ref[0] + acc_ref[1]
    xv = xv_ref[...]
    a2 = jnp.concatenate([g[:, :H], xv[:, :H]], axis=1)
    b2 = jnp.concatenate([g[:, H:], xv[:, H:]], axis=1)
    tab_ref[...] = _pack_bf16_pair(a2, b2)


def _final_kernel(xw_ref, acc2_ref, b1_ref, w2_ref, b2_ref, out_ref):
    s = acc2_ref[0] + acc2_ref[1]
    pre = xw_ref[...] + s[:, :H] + s[:, H:] + b1_ref[...]
    out_ref[...] = (
        jnp.dot(jnp.maximum(pre, 0.0), w2_ref[...],
                preferred_element_type=jnp.float32)
        + b2_ref[...]
    )


@functools.partial(
    pl.kernel,
    out_type=jax.ShapeDtypeStruct((NC * N, OW), jnp.float32),
    mesh=_mesh,
    scratch_types=[
        pltpu.VMEM((4, 4, C), jnp.int32),
        pltpu.VMEM((2, C, GWI), jnp.int32),
        pltpu.VMEM((2, C, OW), jnp.float32),
        pltpu.VMEM_SHARED((N, OW), jnp.float32),
        pltpu.SemaphoreType.DMA,
        pltpu.SemaphoreType.DMA,
        pltpu.SemaphoreType.DMA,
        pltpu.SemaphoreType.DMA,
        pltpu.SemaphoreType.DMA,
        pltpu.SemaphoreType.DMA,
        pltpu.SemaphoreType.DMA,
        pltpu.SemaphoreType.DMA,
    ],
)
def _sc_stage(tab_hbm, meta_hbm, out_hbm, meta_v, rows_v, orow_v, acc_sh,
              m0, m1, m2, m3, g0, g1, s0, s1):
    msem = (m0, m1, m2, m3)
    gsem = (g0, g1)
    ssem = (s0, s1)
    cid = lax.axis_index("c")
    sid = lax.axis_index("s")
    wid = cid * NS + sid
    mbase = wid * CH

    def meta_issue(i, slot):
        pltpu.async_copy(meta_hbm.at[mbase + i], meta_v.at[slot], msem[slot])

    def meta_wait(slot):
        pltpu.make_async_copy(
            meta_hbm.at[mbase], meta_v.at[slot], msem[slot]).wait()

    def gather_issue(slot, b):
        pltpu.async_copy(
            tab_hbm.at[meta_v.at[slot].at[0]], rows_v.at[b], gsem[b])

    def gather_wait(slot, b):
        pltpu.make_async_copy(
            tab_hbm.at[meta_v.at[slot].at[0]], rows_v.at[b], gsem[b]).wait()

    def scatter_issue(slot, b):
        pltpu.async_copy(
            orow_v.at[b],
            acc_sh.at[meta_v.at[slot].at[1]], ssem[b], add=True)

    def scatter_wait(slot, b):
        pltpu.make_async_copy(
            orow_v.at[b],
            acc_sh.at[meta_v.at[slot].at[1]], ssem[b]).wait()

    def compute(slot, b):
        def grp(g, _):
            w0f = lax.bitcast_convert_type(meta_v[slot, 2, pl.ds(g * 16, 16)],
                                           jnp.float32)
            w1f = lax.bitcast_convert_type(meta_v[slot, 3, pl.ds(g * 16, 16)],
                                           jnp.float32)

            def edge(e16, _):
                e = g * 16 + e16
                lane = jnp.full((16,), e16, jnp.int32)
                w0b = w0f[lane]
                w1b = w1f[lane]
                for c in range(NBLK):
                    v = rows_v[b, e, pl.ds(c * 16, 16)]
                    a = lax.bitcast_convert_type(v << 16, jnp.float32)
                    bb = lax.bitcast_convert_type(v & -65536, jnp.float32)
                    orow_v[b, e, pl.ds(c * 16, 16)] = a * w0b + bb * w1b
                return 0

            lax.fori_loop(0, 16, edge, 0)
            return 0

        lax.fori_loop(0, C // 16, grp, 0)

    zv = jnp.zeros((16,), jnp.float32)

    def zrow(e, _):
        for c in range(NBLK):
            orow_v[0, e, pl.ds(c * 16, 16)] = zv
        return 0

    lax.fori_loop(0, C, zrow, 0)
    s625 = sid * (N // NS)
    start = jnp.minimum((s625 // 8) * 8, N - STRIPE)
    start = pl.multiple_of(start, 8)
    for j in range(STRIPE // C):
        pltpu.sync_copy(orow_v.at[0],
                        acc_sh.at[pl.ds(start + j * C, C)])
    plsc.subcore_barrier()

    meta_issue(0, 0)
    meta_issue(1, 1)
    meta_wait(0)
    gather_issue(0, 0)

    meta_wait(1)
    gather_issue(1, 1)
    meta_issue(2, 2)
    gather_wait(0, 0)
    compute(0, 0)
    scatter_issue(0, 0)

    meta_wait(2)
    gather_issue(2, 0)
    meta_issue(3, 3)
    gather_wait(1, 1)
    compute(1, 1)
    scatter_issue(1, 1)

    def body(gi, _):
        i0 = 4 * gi + 2
        for k in range(4):
            i = i0 + k
            slot = (2 + k) % 4
            b = (2 + k) % 2
            nslot = (slot + 1) % 4
            fslot = (slot + 2) % 4
            meta_wait(nslot)
            gather_issue(nslot, b ^ 1)
            scatter_wait(fslot, b)
            meta_issue(i + 2, fslot)
            gather_wait(slot, b)
            compute(slot, b)
            scatter_issue(slot, b)
        return 0

    lax.fori_loop(0, (CH - 5) // 4, body, 0)

    meta_wait(3)
    gather_issue(3, 1)
    scatter_wait(0, 0)
    meta_issue(CH - 1, 0)
    gather_wait(2, 0)
    compute(2, 0)
    scatter_issue(2, 0)

    meta_wait(0)
    gather_issue(0, 0)
    scatter_wait(1, 1)
    gather_wait(3, 1)
    compute(3, 1)
    scatter_issue(3, 1)

    scatter_wait(2, 0)
    gather_wait(0, 0)
    compute(0, 0)
    scatter_issue(0, 0)

    scatter_wait(3, 1)
    scatter_wait(0, 0)
    plsc.subcore_barrier()

    pltpu.sync_copy(
        acc_sh.at[pl.ds(start, STRIPE)],
        out_hbm.at[pl.ds(cid * N + start, STRIPE)])


def kernel(x, edge_index, kernel_w, W1, b1, W2, b2):
    W1x = W1[0:D]
    V0, V1 = W1[D:2 * D], W1[2 * D:3 * D]
    U00, U01 = W1[3 * D:4 * D], W1[4 * D:5 * D]
    U10, U11 = W1[5 * D:6 * D], W1[6 * D:7 * D]
    M = jnp.concatenate([W1x, U00, U10, U01, U11, V0, V1], axis=1)

    meta = jnp.stack(
        [edge_index[0].reshape(NW * CH, C),
         edge_index[1].reshape(NW * CH, C),
         lax.bitcast_convert_type(kernel_w[0], jnp.int32).reshape(NW * CH, C),
         lax.bitcast_convert_type(kernel_w[1], jnp.int32).reshape(NW * CH, C)],
        axis=1)

    xw1x, ptab, xv = pl.pallas_call(
        _proj_kernel,
        out_shape=[jax.ShapeDtypeStruct((N, H), jnp.float32),
                   jax.ShapeDtypeStruct((N, GWI), jnp.int32),
                   jax.ShapeDtypeStruct((N, OW), jnp.float32)],
    )(x, M)

    acc1 = _sc_stage(ptab, meta).reshape(NC, N, OW)

    tab2 = pl.pallas_call(
        _mid_kernel,
        out_shape=jax.ShapeDtypeStruct((N, GWI), jnp.int32),
    )(acc1, xv)

    acc2 = _sc_stage(tab2, meta).reshape(NC, N, OW)

    emb = pl.pallas_call(
        _final_kernel,
        out_shape=jax.ShapeDtypeStruct((N, OUT), jnp.float32),
    )(xw1x, acc2, b1.reshape(1, H), W2, b2.reshape(1, OUT))
    return emb


# speedup vs baseline: 1.1066x; 1.0066x over previous
import functools

import jax
import jax.numpy as jnp
from jax import lax
from jax.experimental import pallas as pl
from jax.experimental.pallas import tpu as pltpu
from jax.experimental.pallas import tpu_sc as plsc

N = 10000
D = 128
H = 64
OUT = 32

NC = 2
NS = 16
NW = NC * NS
E = 320000
EW = E // NW
C = 80
CH = EW // C
GW = 256
GWI = 128
OW = 128
NBLK = OW // 16
STRIPE = 640

_mesh = plsc.VectorSubcoreMesh(core_axis_name="c", subcore_axis_name="s")


def _pack_bf16_pair(a, b):
    ai = lax.bitcast_convert_type(a.astype(jnp.bfloat16), jnp.int16)
    bi = lax.bitcast_convert_type(b.astype(jnp.bfloat16), jnp.int16)
    return (ai.astype(jnp.int32) & 0xFFFF) | (bi.astype(jnp.int32) << 16)


def _proj_kernel(x_ref, m_ref, xw_ref, ptab_ref, xv_ref):
    p = jnp.dot(x_ref[...], m_ref[...], preferred_element_type=jnp.float32)
    xw_ref[...] = p[:, :H]
    ptab_ref[...] = _pack_bf16_pair(p[:, H:H + GWI], p[:, H + GWI:H + GW])
    xv_ref[...] = p[:, H + GW:]


def _mid_kernel(acc_ref, xv_ref, tab_ref):
    g = acc_ref[0] + acc_ref[1]
    xv = xv_ref[...]
    a2 = jnp.concatenate([g[:, :H], xv[:, :H]], axis=1)
    b2 = jnp.concatenate([g[:, H:], xv[:, H:]], axis=1)
    tab_ref[...] = _pack_bf16_pair(a2, b2)


def _final_kernel(xw_ref, acc2_ref, b1_ref, w2_ref, b2_ref, out_ref):
    s = acc2_ref[0] + acc2_ref[1]
    pre = xw_ref[...] + s[:, :H] + s[:, H:] + b1_ref[...]
    out_ref[...] = (
        jnp.dot(jnp.maximum(pre, 0.0), w2_ref[...],
                preferred_element_type=jnp.float32)
        + b2_ref[...]
    )


@functools.partial(
    pl.kernel,
    out_type=jax.ShapeDtypeStruct((NC * N, OW), jnp.float32),
    mesh=_mesh,
    scratch_types=[
        pltpu.VMEM((4, 4, C), jnp.int32),
        pltpu.VMEM((2, C, GWI), jnp.int32),
        pltpu.VMEM((2, C, OW), jnp.float32),
        pltpu.VMEM_SHARED((N, OW), jnp.float32),
        pltpu.SemaphoreType.DMA,
        pltpu.SemaphoreType.DMA,
        pltpu.SemaphoreType.DMA,
        pltpu.SemaphoreType.DMA,
        pltpu.SemaphoreType.DMA,
    ],
)
def _sc_stage(tab_hbm, meta_hbm, out_hbm, meta_v, rows_v, orow_v, acc_sh,
              msem, g0, g1, s0, s1):
    gsem = (g0, g1)
    ssem = (s0, s1)
    cid = lax.axis_index("c")
    sid = lax.axis_index("s")
    wid = cid * NS + sid
    mbase = wid * CH

    def meta_issue(i, slot):
        pltpu.async_copy(meta_hbm.at[mbase + i], meta_v.at[slot], msem)

    def meta_wait(slot):
        pltpu.make_async_copy(
            meta_hbm.at[mbase], meta_v.at[slot], msem).wait()

    def gather_issue(slot, b):
        pltpu.async_copy(
            tab_hbm.at[meta_v.at[slot].at[0]], rows_v.at[b], gsem[b])

    def gather_wait(slot, b):
        pltpu.make_async_copy(
            tab_hbm.at[meta_v.at[slot].at[0]], rows_v.at[b], gsem[b]).wait()

    def scatter_issue(slot, b):
        pltpu.async_copy(
            orow_v.at[b],
            acc_sh.at[meta_v.at[slot].at[1]], ssem[b], add=True)

    def scatter_wait(slot, b):
        pltpu.make_async_copy(
            orow_v.at[b],
            acc_sh.at[meta_v.at[slot].at[1]], ssem[b]).wait()

    def compute(slot, b):
        def grp(sg, _):
            g16 = (sg // 2) * 16
            w0f = lax.bitcast_convert_type(meta_v[slot, 2, pl.ds(g16, 16)],
                                           jnp.float32)
            w1f = lax.bitcast_convert_type(meta_v[slot, 3, pl.ds(g16, 16)],
                                           jnp.float32)
            half = lax.rem(sg, 2) * 8
            base = sg * 8
            for e8 in range(8):
                e = base + e8
                lane = jnp.full((16,), half + e8, jnp.int32)
                w0b = w0f[lane]
                w1b = w1f[lane]
                for c in range(NBLK):
                    v = rows_v[b, e, pl.ds(c * 16, 16)]
                    a = lax.bitcast_convert_type(v << 16, jnp.float32)
                    bb = lax.bitcast_convert_type(v & -65536, jnp.float32)
                    orow_v[b, e, pl.ds(c * 16, 16)] = a * w0b + bb * w1b
            return 0

        lax.fori_loop(0, C // 8, grp, 0)

    zv = jnp.zeros((16,), jnp.float32)

    def zrow(e, _):
        for c in range(NBLK):
            orow_v[0, e, pl.ds(c * 16, 16)] = zv
        return 0

    lax.fori_loop(0, C, zrow, 0)
    s625 = sid * (N // NS)
    start = jnp.minimum((s625 // 8) * 8, N - STRIPE)
    start = pl.multiple_of(start, 8)
    for j in range(STRIPE // C):
        pltpu.sync_copy(orow_v.at[0],
                        acc_sh.at[pl.ds(start + j * C, C)])
    plsc.subcore_barrier()

    meta_issue(0, 0)
    meta_wait(0)
    gather_issue(0, 0)
    meta_issue(1, 1)

    def pair(h, _):
        for k in range(2):
            i = 2 * h + k
            b = k
            slot = lax.rem(i, 4)
            nslot = lax.rem(i + 1, 4)
            fslot = lax.rem(i + 2, 4)
            meta_wait(nslot)
            gather_issue(nslot, b ^ 1)

            @pl.when(i >= 2)
            def _():
                scatter_wait(fslot, b)

            @pl.when(i <= CH - 3)
            def _():
                meta_issue(i + 2, fslot)

            gather_wait(slot, b)
            compute(slot, b)
            scatter_issue(slot, b)
        return 0

    lax.fori_loop(0, (CH - 1) // 2, pair, 0)

    scatter_wait(2, 0)
    gather_wait(0, 0)
    compute(0, 0)
    scatter_issue(0, 0)

    scatter_wait(3, 1)
    scatter_wait(0, 0)
    plsc.subcore_barrier()

    pltpu.sync_copy(
        acc_sh.at[pl.ds(start, STRIPE)],
        out_hbm.at[pl.ds(cid * N + start, STRIPE)])


def kernel(x, edge_index, kernel_w, W1, b1, W2, b2):
    W1x = W1[0:D]
    V0, V1 = W1[D:2 * D], W1[2 * D:3 * D]
    U00, U01 = W1[3 * D:4 * D], W1[4 * D:5 * D]
    U10, U11 = W1[5 * D:6 * D], W1[6 * D:7 * D]
    M = jnp.concatenate([W1x, U00, U10, U01, U11, V0, V1], axis=1)

    meta = jnp.stack(
        [edge_index[0].reshape(NW * CH, C),
         edge_index[1].reshape(NW * CH, C),
         lax.bitcast_convert_type(kernel_w[0], jnp.int32).reshape(NW * CH, C),
         lax.bitcast_convert_type(kernel_w[1], jnp.int32).reshape(NW * CH, C)],
        axis=1)

    xw1x, ptab, xv = pl.pallas_call(
        _proj_kernel,
        out_shape=[jax.ShapeDtypeStruct((N, H), jnp.float32),
                   jax.ShapeDtypeStruct((N, GWI), jnp.int32),
                   jax.ShapeDtypeStruct((N, OW), jnp.float32)],
    )(x, M)

    acc1 = _sc_stage(ptab, meta).reshape(NC, N, OW)

    tab2 = pl.pallas_call(
        _mid_kernel,
        out_shape=jax.ShapeDtypeStruct((N, GWI), jnp.int32),
    )(acc1, xv)

    acc2 = _sc_stage(tab2, meta).reshape(NC, N, OW)

    emb = pl.pallas_call(
        _final_kernel,
        out_shape=jax.ShapeDtypeStruct((N, OUT), jnp.float32),
    )(xw1x, acc2, b1.reshape(1, H), W2, b2.reshape(1, OUT))
    return emb


# speedup vs baseline: 2.3925x; 2.1621x over previous
import functools

import jax
import jax.numpy as jnp
from jax import lax
from jax.experimental import pallas as pl
from jax.experimental.pallas import tpu as pltpu
from jax.experimental.pallas import tpu_sc as plsc

N = 10000
D = 128
H = 64
OUT = 32

NC = 2
NS = 16
NW = NC * NS
E = 320000
EW = E // NW
C = 80
CH = EW // C
GW = 256
GWI = 128
OW = 128
NBLK = OW // 16
STRIPE = 640

_mesh = plsc.VectorSubcoreMesh(core_axis_name="c", subcore_axis_name="s")


def _pack_bf16_pair(a, b):
    ai = lax.bitcast_convert_type(a.astype(jnp.bfloat16), jnp.int16)
    bi = lax.bitcast_convert_type(b.astype(jnp.bfloat16), jnp.int16)
    return (ai.astype(jnp.int32) & 0xFFFF) | (bi.astype(jnp.int32) << 16)


def _proj_kernel(x_ref, m_ref, xw_ref, ptab_ref, xv_ref):
    p = jnp.dot(x_ref[...], m_ref[...], preferred_element_type=jnp.float32)
    xw_ref[...] = p[:, :H]
    ptab_ref[...] = _pack_bf16_pair(p[:, H:H + GWI], p[:, H + GWI:H + GW])
    xv_ref[...] = p[:, H + GW:]


def _mid_kernel(acc_ref, xv_ref, tab_ref):
    g = acc_ref[0] + acc_ref[1]
    xv = xv_ref[...]
    a2 = jnp.concatenate([g[:, :H], xv[:, :H]], axis=1)
    b2 = jnp.concatenate([g[:, H:], xv[:, H:]], axis=1)
    tab_ref[...] = _pack_bf16_pair(a2, b2)


def _final_kernel(xw_ref, acc2_ref, b1_ref, w2_ref, b2_ref, out_ref):
    s = acc2_ref[0] + acc2_ref[1]
    pre = xw_ref[...] + s[:, :H] + s[:, H:] + b1_ref[...]
    out_ref[...] = (
        jnp.dot(jnp.maximum(pre, 0.0), w2_ref[...],
                preferred_element_type=jnp.float32)
        + b2_ref[...]
    )


@functools.partial(
    pl.kernel,
    out_type=jax.ShapeDtypeStruct((NC * N, OW), jnp.float32),
    mesh=_mesh,
    scratch_types=[
        pltpu.VMEM((4, 4, C), jnp.int32),
        pltpu.VMEM((2, C, GWI), jnp.int32),
        pltpu.VMEM((2, C, OW), jnp.float32),
        pltpu.VMEM_SHARED((N, OW), jnp.float32),
        pltpu.SemaphoreType.DMA,
        pltpu.SemaphoreType.DMA,
        pltpu.SemaphoreType.DMA,
        pltpu.SemaphoreType.DMA,
        pltpu.SemaphoreType.DMA,
    ],
)
def _sc_stage(tab_hbm, meta_hbm, out_hbm, meta_v, rows_v, orow_v, acc_sh,
              msem, g0, g1, s0, s1):
    gsem = (g0, g1)
    ssem = (s0, s1)
    cid = lax.axis_index("c")
    sid = lax.axis_index("s")
    wid = cid * NS + sid
    mbase = wid * CH

    def meta_issue(i, slot):
        pltpu.async_copy(meta_hbm.at[mbase + i], meta_v.at[slot], msem)

    def meta_wait(slot):
        pltpu.make_async_copy(
            meta_hbm.at[mbase], meta_v.at[slot], msem).wait()

    def gather_issue(slot, b):
        pltpu.async_copy(
            tab_hbm.at[meta_v.at[slot].at[0]], rows_v.at[b], gsem[b])

    def gather_wait(slot, b):
        pltpu.make_async_copy(
            tab_hbm.at[meta_v.at[slot].at[0]], rows_v.at[b], gsem[b]).wait()

    def scatter_issue(slot, b):
        pltpu.async_copy(
            orow_v.at[b],
            acc_sh.at[meta_v.at[slot].at[1]], ssem[b], add=True)

    def scatter_wait(slot, b):
        pltpu.make_async_copy(
            orow_v.at[b],
            acc_sh.at[meta_v.at[slot].at[1]], ssem[b]).wait()

    ILP = 4

    def compute(slot, b):
        def grp(sg, _):
            g16 = (sg // 2) * 16
            w0f = lax.bitcast_convert_type(meta_v[slot, 2, pl.ds(g16, 16)],
                                           jnp.float32)
            w1f = lax.bitcast_convert_type(meta_v[slot, 3, pl.ds(g16, 16)],
                                           jnp.float32)
            half = lax.rem(sg, 2) * 8
            base = sg * 8
            for q in range(8 // ILP):
                es = [base + q * ILP + j for j in range(ILP)]
                w0bs = [w0f[jnp.full((16,), half + q * ILP + j, jnp.int32)]
                        for j in range(ILP)]
                w1bs = [w1f[jnp.full((16,), half + q * ILP + j, jnp.int32)]
                        for j in range(ILP)]
                for c in range(NBLK):
                    vs = [rows_v[b, e, pl.ds(c * 16, 16)] for e in es]
                    az = [lax.bitcast_convert_type(v << 16, jnp.float32)
                          for v in vs]
                    bz = [lax.bitcast_convert_type(v & -65536, jnp.float32)
                          for v in vs]
                    oz = [a * w0 + bb * w1
                          for a, bb, w0, w1 in zip(az, bz, w0bs, w1bs)]
                    for j in range(ILP):
                        orow_v[b, es[j], pl.ds(c * 16, 16)] = oz[j]
            return 0

        lax.fori_loop(0, C // 8, grp, 0)

    zv = jnp.zeros((16,), jnp.float32)

    def zrow(e, _):
        for c in range(NBLK):
            orow_v[0, e, pl.ds(c * 16, 16)] = zv
        return 0

    lax.fori_loop(0, C, zrow, 0)
    s625 = sid * (N // NS)
    start = jnp.minimum((s625 // 8) * 8, N - STRIPE)
    start = pl.multiple_of(start, 8)
    for j in range(STRIPE // C):
        pltpu.sync_copy(orow_v.at[0],
                        acc_sh.at[pl.ds(start + j * C, C)])
    plsc.subcore_barrier()

    meta_issue(0, 0)
    meta_wait(0)
    gather_issue(0, 0)
    meta_issue(1, 1)

    def pair(h, _):
        for k in range(2):
            i = 2 * h + k
            b = k
            slot = lax.rem(i, 4)
            nslot = lax.rem(i + 1, 4)
            fslot = lax.rem(i + 2, 4)
            meta_wait(nslot)
            gather_issue(nslot, b ^ 1)

            @pl.when(i >= 2)
            def _():
                scatter_wait(fslot, b)

            @pl.when(i <= CH - 3)
            def _():
                meta_issue(i + 2, fslot)

            gather_wait(slot, b)
            compute(slot, b)
            scatter_issue(slot, b)
        return 0

    lax.fori_loop(0, (CH - 1) // 2, pair, 0)

    scatter_wait(2, 0)
    gather_wait(0, 0)
    compute(0, 0)
    scatter_issue(0, 0)

    scatter_wait(3, 1)
    scatter_wait(0, 0)
    plsc.subcore_barrier()

    pltpu.sync_copy(
        acc_sh.at[pl.ds(start, STRIPE)],
        out_hbm.at[pl.ds(cid * N + start, STRIPE)])


def kernel(x, edge_index, kernel_w, W1, b1, W2, b2):
    W1x = W1[0:D]
    V0, V1 = W1[D:2 * D], W1[2 * D:3 * D]
    U00, U01 = W1[3 * D:4 * D], W1[4 * D:5 * D]
    U10, U11 = W1[5 * D:6 * D], W1[6 * D:7 * D]
    M = jnp.concatenate([W1x, U00, U10, U01, U11, V0, V1], axis=1)

    meta = jnp.stack(
        [edge_index[0].reshape(NW * CH, C),
         edge_index[1].reshape(NW * CH, C),
         lax.bitcast_convert_type(kernel_w[0], jnp.int32).reshape(NW * CH, C),
         lax.bitcast_convert_type(kernel_w[1], jnp.int32).reshape(NW * CH, C)],
        axis=1)

    xw1x, ptab, xv = pl.pallas_call(
        _proj_kernel,
        out_shape=[jax.ShapeDtypeStruct((N, H), jnp.float32),
                   jax.ShapeDtypeStruct((N, GWI), jnp.int32),
                   jax.ShapeDtypeStruct((N, OW), jnp.float32)],
    )(x, M)

    acc1 = _sc_stage(ptab, meta).reshape(NC, N, OW)

    tab2 = pl.pallas_call(
        _mid_kernel,
        out_shape=jax.ShapeDtypeStruct((N, GWI), jnp.int32),
    )(acc1, xv)

    acc2 = _sc_stage(tab2, meta).reshape(NC, N, OW)

    emb = pl.pallas_call(
        _final_kernel,
        out_shape=jax.ShapeDtypeStruct((N, OUT), jnp.float32),
    )(xw1x, acc2, b1.reshape(1, H), W2, b2.reshape(1, OUT))
    return emb


# speedup vs baseline: 2.7819x; 1.1627x over previous
import functools

import jax
import jax.numpy as jnp
from jax import lax
from jax.experimental import pallas as pl
from jax.experimental.pallas import tpu as pltpu
from jax.experimental.pallas import tpu_sc as plsc

N = 10000
D = 128
H = 64
OUT = 32

NC = 2
NS = 16
NW = NC * NS
E = 320000
EW = E // NW
C = 80
CH = EW // C
GW = 256
GWI = 128
OW = 128
NBLK = OW // 16
STRIPE = 640

_mesh = plsc.VectorSubcoreMesh(core_axis_name="c", subcore_axis_name="s")


def _pack_bf16_pair(a, b):
    ai = lax.bitcast_convert_type(a.astype(jnp.bfloat16), jnp.int16)
    bi = lax.bitcast_convert_type(b.astype(jnp.bfloat16), jnp.int16)
    return (ai.astype(jnp.int32) & 0xFFFF) | (bi.astype(jnp.int32) << 16)


def _proj_kernel(x_ref, m_ref, xw_ref, ptab_ref, xv_ref):
    p = jnp.dot(x_ref[...], m_ref[...], preferred_element_type=jnp.float32)
    xw_ref[...] = p[:, :H]
    ptab_ref[...] = _pack_bf16_pair(p[:, H:H + GWI], p[:, H + GWI:H + GW])
    xv_ref[...] = p[:, H + GW:]


def _mid_kernel(acc_ref, xv_ref, tab_ref):
    g = acc_ref[0] + acc_ref[1]
    xv = xv_ref[...]
    a2 = jnp.concatenate([g[:, :H], xv[:, :H]], axis=1)
    b2 = jnp.concatenate([g[:, H:], xv[:, H:]], axis=1)
    tab_ref[...] = _pack_bf16_pair(a2, b2)


def _final_kernel(xw_ref, acc2_ref, b1_ref, w2_ref, b2_ref, out_ref):
    s = acc2_ref[0] + acc2_ref[1]
    pre = xw_ref[...] + s[:, :H] + s[:, H:] + b1_ref[...]
    out_ref[...] = (
        jnp.dot(jnp.maximum(pre, 0.0), w2_ref[...],
                preferred_element_type=jnp.float32)
        + b2_ref[...]
    )


@functools.partial(
    pl.kernel,
    out_type=jax.ShapeDtypeStruct((NC * N, OW), jnp.float32),
    mesh=_mesh,
    scratch_types=[
        pltpu.VMEM((4, 4, C), jnp.int32),
        pltpu.VMEM((2, C, GWI), jnp.int32),
        pltpu.VMEM((2, C, OW), jnp.float32),
        pltpu.VMEM_SHARED((N, OW), jnp.float32),
        pltpu.SemaphoreType.DMA,
        pltpu.SemaphoreType.DMA,
        pltpu.SemaphoreType.DMA,
        pltpu.SemaphoreType.DMA,
        pltpu.SemaphoreType.DMA,
    ],
)
def _sc_stage(tab_hbm, meta_hbm, out_hbm, meta_v, rows_v, orow_v, acc_sh,
              msem, g0, g1, s0, s1):
    gsem = (g0, g1)
    ssem = (s0, s1)
    cid = lax.axis_index("c")
    sid = lax.axis_index("s")
    wid = cid * NS + sid
    mbase = wid * CH

    def meta_issue(i, slot):
        pltpu.async_copy(meta_hbm.at[mbase + i], meta_v.at[slot], msem)

    def meta_wait(slot):
        pltpu.make_async_copy(
            meta_hbm.at[mbase], meta_v.at[slot], msem).wait()

    def gather_issue(slot, b):
        pltpu.async_copy(
            tab_hbm.at[meta_v.at[slot].at[0]], rows_v.at[b], gsem[b])

    def gather_wait(slot, b):
        pltpu.make_async_copy(
            tab_hbm.at[meta_v.at[slot].at[0]], rows_v.at[b], gsem[b]).wait()

    def scatter_issue(slot, b):
        pltpu.async_copy(
            orow_v.at[b],
            acc_sh.at[meta_v.at[slot].at[1]], ssem[b], add=True)

    def scatter_wait(slot, b):
        pltpu.make_async_copy(
            orow_v.at[b],
            acc_sh.at[meta_v.at[slot].at[1]], ssem[b]).wait()

    ILP = 8

    def compute(slot, b):
        def grp(sg, _):
            g16 = (sg // 2) * 16
            w0f = lax.bitcast_convert_type(meta_v[slot, 2, pl.ds(g16, 16)],
                                           jnp.float32)
            w1f = lax.bitcast_convert_type(meta_v[slot, 3, pl.ds(g16, 16)],
                                           jnp.float32)
            half = lax.rem(sg, 2) * 8
            base = sg * 8
            for q in range(8 // ILP):
                es = [base + q * ILP + j for j in range(ILP)]
                w0bs = [w0f[jnp.full((16,), half + q * ILP + j, jnp.int32)]
                        for j in range(ILP)]
                w1bs = [w1f[jnp.full((16,), half + q * ILP + j, jnp.int32)]
                        for j in range(ILP)]
                for c in range(NBLK):
                    vs = [rows_v[b, e, pl.ds(c * 16, 16)] for e in es]
                    az = [lax.bitcast_convert_type(v << 16, jnp.float32)
                          for v in vs]
                    bz = [lax.bitcast_convert_type(v & -65536, jnp.float32)
                          for v in vs]
                    oz = [a * w0 + bb * w1
                          for a, bb, w0, w1 in zip(az, bz, w0bs, w1bs)]
                    for j in range(ILP):
                        orow_v[b, es[j], pl.ds(c * 16, 16)] = oz[j]
            return 0

        lax.fori_loop(0, C // 8, grp, 0)

    zv = jnp.zeros((16,), jnp.float32)

    def zrow(e, _):
        for c in range(NBLK):
            orow_v[0, e, pl.ds(c * 16, 16)] = zv
        return 0

    lax.fori_loop(0, C, zrow, 0)
    s625 = sid * (N // NS)
    start = jnp.minimum((s625 // 8) * 8, N - STRIPE)
    start = pl.multiple_of(start, 8)
    for j in range(STRIPE // C):
        pltpu.sync_copy(orow_v.at[0],
                        acc_sh.at[pl.ds(start + j * C, C)])
    plsc.subcore_barrier()

    meta_issue(0, 0)
    meta_wait(0)
    gather_issue(0, 0)
    meta_issue(1, 1)

    def pair(h, _):
        for k in range(2):
            i = 2 * h + k
            b = k
            slot = lax.rem(i, 4)
            nslot = lax.rem(i + 1, 4)
            fslot = lax.rem(i + 2, 4)
            meta_wait(nslot)
            gather_issue(nslot, b ^ 1)

            @pl.when(i >= 2)
            def _():
                scatter_wait(fslot, b)

            @pl.when(i <= CH - 3)
            def _():
                meta_issue(i + 2, fslot)

            gather_wait(slot, b)
            compute(slot, b)
            scatter_issue(slot, b)
        return 0

    lax.fori_loop(0, (CH - 1) // 2, pair, 0)

    scatter_wait(2, 0)
    gather_wait(0, 0)
    compute(0, 0)
    scatter_issue(0, 0)

    scatter_wait(3, 1)
    scatter_wait(0, 0)
    plsc.subcore_barrier()

    pltpu.sync_copy(
        acc_sh.at[pl.ds(start, STRIPE)],
        out_hbm.at[pl.ds(cid * N + start, STRIPE)])


def kernel(x, edge_index, kernel_w, W1, b1, W2, b2):
    W1x = W1[0:D]
    V0, V1 = W1[D:2 * D], W1[2 * D:3 * D]
    U00, U01 = W1[3 * D:4 * D], W1[4 * D:5 * D]
    U10, U11 = W1[5 * D:6 * D], W1[6 * D:7 * D]
    M = jnp.concatenate([W1x, U00, U10, U01, U11, V0, V1], axis=1)

    meta = jnp.stack(
        [edge_index[0].reshape(NW * CH, C),
         edge_index[1].reshape(NW * CH, C),
         lax.bitcast_convert_type(kernel_w[0], jnp.int32).reshape(NW * CH, C),
         lax.bitcast_convert_type(kernel_w[1], jnp.int32).reshape(NW * CH, C)],
        axis=1)

    xw1x, ptab, xv = pl.pallas_call(
        _proj_kernel,
        out_shape=[jax.ShapeDtypeStruct((N, H), jnp.float32),
                   jax.ShapeDtypeStruct((N, GWI), jnp.int32),
                   jax.ShapeDtypeStruct((N, OW), jnp.float32)],
    )(x, M)

    acc1 = _sc_stage(ptab, meta).reshape(NC, N, OW)

    tab2 = pl.pallas_call(
        _mid_kernel,
        out_shape=jax.ShapeDtypeStruct((N, GWI), jnp.int32),
    )(acc1, xv)

    acc2 = _sc_stage(tab2, meta).reshape(NC, N, OW)

    emb = pl.pallas_call(
        _final_kernel,
        out_shape=jax.ShapeDtypeStruct((N, OUT), jnp.float32),
    )(xw1x, acc2, b1.reshape(1, H), W2, b2.reshape(1, OUT))
    return emb
